# baseline (speedup 1.0000x reference)
import functools

import jax
import jax.numpy as jnp
from jax import lax
from jax.experimental import pallas as pl
from jax.experimental.pallas import tpu as pltpu
from jax.experimental.pallas import tpu_sc as plsc

T = 2048
D = 1024
F = 2048
E = 8
K = 2
P = T * K
BLK = 128
NB = P // BLK + E
NP = NB * BLK

NC = 2
NS = 16
NW = NC * NS

D2 = D // 2
GROWS = NP // NW
GSUB = 32
NSUB = GROWS // GSUB


@functools.cache
def _sc_gather_rows():
    mesh = plsc.VectorSubcoreMesh(core_axis_name="c", subcore_axis_name="s")

    @functools.partial(
        pl.kernel,
        out_type=jax.ShapeDtypeStruct((NP, D2), jnp.int32),
        mesh=mesh,
        scratch_types=[
            pltpu.VMEM((NSUB, GSUB), jnp.int32),
            pltpu.VMEM((NSUB, GSUB, D2), jnp.int32),
            pltpu.SemaphoreType.DMA((NSUB,)),
            pltpu.SemaphoreType.DMA((NSUB,)),
        ],
    )
    def gather_k(table_hbm, idx_hbm, out_hbm, idx_v, bufs, gsem, ssem):
        wid = lax.axis_index("s") * NC + lax.axis_index("c")
        base = wid * GROWS
        for c in range(NSUB):
            pltpu.sync_copy(
                idx_hbm.at[pl.ds(base + c * GSUB, GSUB)], idx_v.at[c]
            )
        gcs = [
            pltpu.async_copy(
                table_hbm.at[idx_v.at[c]], bufs.at[c], gsem.at[c],
            )
            for c in range(NSUB)
        ]
        wcs = []
        for c in range(NSUB):
            gcs[c].wait()
            wcs.append(
                pltpu.async_copy(
                    bufs.at[c], out_hbm.at[pl.ds(base + c * GSUB, GSUB)],
                    ssem.at[c],
                )
            )
        for w in wcs:
            w.wait()

    return gather_k


CTOK = T // NW
CCH = 16
NCH = CTOK // CCH


@functools.cache
def _sc_combine():
    mesh = plsc.VectorSubcoreMesh(core_axis_name="c", subcore_axis_name="s")

    @functools.partial(
        pl.kernel,
        out_type=jax.ShapeDtypeStruct((T, D), jnp.float32),
        mesh=mesh,
        scratch_types=[
            pltpu.VMEM((NCH, CCH), jnp.int32),
            pltpu.VMEM((NCH, CCH), jnp.int32),
            pltpu.VMEM((2, CCH, D), jnp.float32),
            pltpu.VMEM((2, CCH, D), jnp.float32),
            pltpu.SemaphoreType.DMA((2,)),
            pltpu.SemaphoreType.DMA((2,)),
            pltpu.SemaphoreType.DMA((2,)),
        ],
    )
    def combine_k(y_hbm, p01_hbm, out_hbm, i0_v, i1_v, r0_v, r1_v, g0s, g1s, wbs):
        wid = lax.axis_index("s") * NC + lax.axis_index("c")
        base = wid * CTOK
        pltpu.sync_copy(p01_hbm.at[0, wid], i0_v)
        pltpu.sync_copy(p01_hbm.at[1, wid], i1_v)

        def _fire(c):
            b = c % 2
            return (
                pltpu.async_copy(y_hbm.at[i0_v.at[c]], r0_v.at[b], g0s.at[b]),
                pltpu.async_copy(y_hbm.at[i1_v.at[c]], r1_v.at[b], g1s.at[b]),
            )

        gp = {0: _fire(0)}
        wb = [None, None]
        for c in range(NCH):
            b = c % 2
            gp[c][0].wait()
            gp[c][1].wait()
            if c + 1 < NCH:
                nb = (c + 1) % 2
                if wb[nb] is not None:
                    wb[nb].wait()
                    wb[nb] = None
                gp[c + 1] = _fire(c + 1)

            def _row(r, _):
                for j in range(D // 16):
                    sl = pl.ds(j * 16, 16)
                    r0_v[b, r, sl] = r0_v[b, r, sl] + r1_v[b, r, sl]
                return 0

            lax.fori_loop(0, CCH, _row, 0)
            wb[b] = pltpu.async_copy(
                r0_v.at[b], out_hbm.at[pl.ds(base + c * CCH, CCH)], wbs.at[b]
            )
        for b in range(2):
            if wb[b] is not None:
                wb[b].wait()

    return combine_k


def _mlp_block_kernel(
    meta_ref, x_ref, w1_hbm, w2_hbm, g_ref, y_ref, w1_buf, w2_buf, sems
):
    i = pl.program_id(0)
    e = meta_ref[0, i]
    slot = meta_ref[1, i]
    first = meta_ref[2, i]
    has_next = meta_ref[3, i]
    nxt_e = meta_ref[4, i]

    def _w_copies(expert, s):
        c1 = pltpu.make_async_copy(w1_hbm.at[expert], w1_buf.at[s], sems.at[s, 0])
        c2 = pltpu.make_async_copy(w2_hbm.at[expert], w2_buf.at[s], sems.at[s, 1])
        return c1, c2

    @pl.when(i == 0)
    def _prologue():
        c1, c2 = _w_copies(e, slot)
        c1.start()
        c2.start()

    @pl.when((first == 1) & (has_next == 1))
    def _prefetch_next():
        c1, c2 = _w_copies(nxt_e, 1 - slot)
        c1.start()
        c2.start()

    @pl.when(first == 1)
    def _wait_cur():
        c1, c2 = _w_copies(e, slot)
        c1.wait()
        c2.wait()

    xi = x_ref[...]
    x_lo = pltpu.bitcast(xi << 16, jnp.float32).astype(jnp.bfloat16)
    x_hi = pltpu.bitcast(xi & jnp.int32(-65536), jnp.float32).astype(jnp.bfloat16)
    w1a = w1_buf[slot, : D2].astype(jnp.bfloat16)
    w1b = w1_buf[slot, D2:].astype(jnp.bfloat16)
    h = jnp.dot(x_lo, w1a, preferred_element_type=jnp.float32) + jnp.dot(
        x_hi, w1b, preferred_element_type=jnp.float32
    )
    h = jax.nn.gelu(h)
    w2 = w2_buf[slot].astype(jnp.bfloat16)
    y = jnp.dot(h.astype(jnp.bfloat16), w2, preferred_element_type=jnp.float32)
    gb = jnp.broadcast_to(g_ref[0], (BLK, BLK))
    ir = lax.broadcasted_iota(jnp.int32, (BLK, BLK), 0)
    ic = lax.broadcasted_iota(jnp.int32, (BLK, BLK), 1)
    diag = jnp.where(ir == ic, gb, jnp.zeros_like(gb))
    y_ref[...] = jnp.dot(diag, y, preferred_element_type=jnp.float32)


def _grouped_mlp(block_meta, x_s, w_fc, w_proj, gates3):
    grid_spec = pltpu.PrefetchScalarGridSpec(
        num_scalar_prefetch=1,
        grid=(NB,),
        in_specs=[
            pl.BlockSpec((BLK, D2), lambda i, m: (i, 0)),
            pl.BlockSpec(memory_space=pl.ANY),
            pl.BlockSpec(memory_space=pl.ANY),
            pl.BlockSpec((1, 1, BLK), lambda i, m: (i, 0, 0)),
        ],
        out_specs=pl.BlockSpec((BLK, D), lambda i, m: (i, 0)),
        scratch_shapes=[
            pltpu.VMEM((2, D, F), jnp.float32),
            pltpu.VMEM((2, F, D), jnp.float32),
            pltpu.SemaphoreType.DMA((2, 2)),
        ],
    )
    return pl.pallas_call(
        _mlp_block_kernel,
        grid_spec=grid_spec,
        out_shape=jax.ShapeDtypeStruct((NP, D), jnp.float32),
        compiler_params=pltpu.CompilerParams(
            dimension_semantics=("arbitrary",),
        ),
    )(block_meta, x_s, w_fc, w_proj, gates3)


def _dispatch_meta(eids, gflat):
    onehot = (eids[None, :] == jnp.arange(E, dtype=jnp.int32)[:, None]).astype(
        jnp.int32
    )
    incl = jnp.cumsum(onehot, axis=1)
    counts = incl[:, -1]
    ranks = jnp.take_along_axis(incl, eids[None, :], axis=0)[0] - 1
    nblk = (counts + BLK - 1) // BLK
    off = jnp.concatenate(
        [jnp.zeros((1,), jnp.int32), jnp.cumsum(nblk * BLK)[:-1].astype(jnp.int32)]
    )
    pos = off[eids] + ranks

    p_token = jnp.arange(P, dtype=jnp.int32) // K
    row_token = jnp.zeros((NP,), jnp.int32).at[pos].set(p_token)
    row_gate = jnp.zeros((NP,), jnp.float32).at[pos].set(gflat)
    be = jnp.repeat(jnp.arange(E, dtype=jnp.int32), nblk, total_repeat_length=NB)
    first = jnp.concatenate(
        [jnp.ones((1,), jnp.int32), (be[1:] != be[:-1]).astype(jnp.int32)]
    )
    run_id = jnp.cumsum(first) - 1
    par = (run_id % 2).astype(jnp.int32)
    idx_end = jnp.searchsorted(be, be, side="right").astype(jnp.int32)
    has_next = (idx_end < NB).astype(jnp.int32)
    nxt_e = be[jnp.minimum(idx_end, NB - 1)]
    block_meta = jnp.stack([be, par, first, has_next, nxt_e]).astype(jnp.int32)
    pos01 = pos.reshape(T, K).T.reshape(K, NW, NCH, CCH)
    gates3 = row_gate.reshape(NB, 1, BLK)
    return row_token, gates3, block_meta, pos01


def kernel(hidden_states, routing_weights, selected_experts, W_fc, W_proj):
    eids = selected_experts.reshape(-1).astype(jnp.int32)
    gflat = routing_weights.reshape(-1).astype(jnp.float32)
    row_token, gates3, block_meta, pos01 = _dispatch_meta(eids, gflat)

    hu = jax.lax.bitcast_convert_type(
        hidden_states.astype(jnp.bfloat16), jnp.uint16
    ).astype(jnp.uint32)
    h_pk = jax.lax.bitcast_convert_type(
        hu[:, :D2] | (hu[:, D2:] << 16), jnp.int32
    )
    x_s = _sc_gather_rows()(h_pk, row_token)
    y = _grouped_mlp(block_meta, x_s, W_fc, W_proj, gates3)
    out = _sc_combine()(y, pos01)
    return out


# speedup vs baseline: 1.0424x; 1.0424x over previous
import functools

import jax
import jax.numpy as jnp
from jax import lax
from jax.experimental import pallas as pl
from jax.experimental.pallas import tpu as pltpu
from jax.experimental.pallas import tpu_sc as plsc

T = 2048
D = 1024
F = 2048
E = 8
K = 2
P = T * K
BLK = 128
NB = P // BLK + E
NP = NB * BLK

NC = 2
NS = 16
NW = NC * NS

D2 = D // 2
GROWS = NP // NW
GSUB = 32
NSUB = GROWS // GSUB


@functools.cache
def _sc_gather_rows():
    mesh = plsc.VectorSubcoreMesh(core_axis_name="c", subcore_axis_name="s")

    @functools.partial(
        pl.kernel,
        out_type=jax.ShapeDtypeStruct((NP, D2), jnp.int32),
        mesh=mesh,
        scratch_types=[
            pltpu.VMEM((NSUB, GSUB), jnp.int32),
            pltpu.VMEM((NSUB, GSUB, D2), jnp.int32),
            pltpu.SemaphoreType.DMA((NSUB,)),
            pltpu.SemaphoreType.DMA((NSUB,)),
        ],
    )
    def gather_k(table_hbm, idx_hbm, out_hbm, idx_v, bufs, gsem, ssem):
        wid = lax.axis_index("s") * NC + lax.axis_index("c")
        base = wid * GROWS
        for c in range(NSUB):
            pltpu.sync_copy(
                idx_hbm.at[pl.ds(base + c * GSUB, GSUB)], idx_v.at[c]
            )
        gcs = [
            pltpu.async_copy(
                table_hbm.at[idx_v.at[c]], bufs.at[c], gsem.at[c],
            )
            for c in range(NSUB)
        ]
        wcs = []
        for c in range(NSUB):
            gcs[c].wait()
            wcs.append(
                pltpu.async_copy(
                    bufs.at[c], out_hbm.at[pl.ds(base + c * GSUB, GSUB)],
                    ssem.at[c],
                )
            )
        for w in wcs:
            w.wait()

    return gather_k


CTOK = T // NW
CCH = 16
NCH = CTOK // CCH


@functools.cache
def _sc_combine():
    mesh = plsc.VectorSubcoreMesh(core_axis_name="c", subcore_axis_name="s")

    @functools.partial(
        pl.kernel,
        out_type=jax.ShapeDtypeStruct((T, D), jnp.float32),
        mesh=mesh,
        scratch_types=[
            pltpu.VMEM((NCH, CCH), jnp.int32),
            pltpu.VMEM((NCH, CCH), jnp.int32),
            pltpu.VMEM((2, CCH, D), jnp.float32),
            pltpu.VMEM((2, CCH, D), jnp.float32),
            pltpu.SemaphoreType.DMA((2,)),
            pltpu.SemaphoreType.DMA((2,)),
            pltpu.SemaphoreType.DMA((2,)),
        ],
    )
    def combine_k(y_hbm, p01_hbm, out_hbm, i0_v, i1_v, r0_v, r1_v, g0s, g1s, wbs):
        wid = lax.axis_index("s") * NC + lax.axis_index("c")
        base = wid * CTOK
        pltpu.sync_copy(p01_hbm.at[0, wid], i0_v)
        pltpu.sync_copy(p01_hbm.at[1, wid], i1_v)

        def _fire(c):
            b = c % 2
            return (
                pltpu.async_copy(y_hbm.at[i0_v.at[c]], r0_v.at[b], g0s.at[b]),
                pltpu.async_copy(y_hbm.at[i1_v.at[c]], r1_v.at[b], g1s.at[b]),
            )

        gp = {0: _fire(0)}
        wb = [None, None]
        for c in range(NCH):
            b = c % 2
            gp[c][0].wait()
            gp[c][1].wait()
            if c + 1 < NCH:
                nb = (c + 1) % 2
                if wb[nb] is not None:
                    wb[nb].wait()
                    wb[nb] = None
                gp[c + 1] = _fire(c + 1)

            def _row(r, _):
                for j in range(D // 16):
                    sl = pl.ds(j * 16, 16)
                    r0_v[b, r, sl] = r0_v[b, r, sl] + r1_v[b, r, sl]
                return 0

            lax.fori_loop(0, CCH, _row, 0)
            wb[b] = pltpu.async_copy(
                r0_v.at[b], out_hbm.at[pl.ds(base + c * CCH, CCH)], wbs.at[b]
            )
        for b in range(2):
            if wb[b] is not None:
                wb[b].wait()

    return combine_k


def _mlp_block_kernel(
    meta_ref, x_ref, w1_hbm, w2_hbm, g_ref, y_ref, w1_buf, w2_buf, sems
):
    i = pl.program_id(0)
    e = meta_ref[0, i]
    slot = meta_ref[1, i]
    first = meta_ref[2, i]
    has_next = meta_ref[3, i]
    nxt_e = meta_ref[4, i]

    def _w_copies(expert, s):
        c1 = pltpu.make_async_copy(w1_hbm.at[expert], w1_buf.at[s], sems.at[s, 0])
        c2 = pltpu.make_async_copy(w2_hbm.at[expert], w2_buf.at[s], sems.at[s, 1])
        return c1, c2

    @pl.when(i == 0)
    def _prologue():
        c1, c2 = _w_copies(e, slot)
        c1.start()
        c2.start()

    @pl.when((first == 1) & (has_next == 1))
    def _prefetch_next():
        c1, c2 = _w_copies(nxt_e, 1 - slot)
        c1.start()
        c2.start()

    @pl.when(first == 1)
    def _wait_cur():
        c1, c2 = _w_copies(e, slot)
        c1.wait()
        c2.wait()

    xi = x_ref[...]
    x_lo = pltpu.bitcast(xi << 16, jnp.float32).astype(jnp.bfloat16)
    x_hi = pltpu.bitcast(xi & jnp.int32(-65536), jnp.float32).astype(jnp.bfloat16)
    w1a = w1_buf[slot, : D2].astype(jnp.bfloat16)
    w1b = w1_buf[slot, D2:].astype(jnp.bfloat16)
    h = jnp.dot(x_lo, w1a, preferred_element_type=jnp.float32) + jnp.dot(
        x_hi, w1b, preferred_element_type=jnp.float32
    )
    h = jax.nn.gelu(h)
    w2 = w2_buf[slot].astype(jnp.bfloat16)
    y = jnp.dot(h.astype(jnp.bfloat16), w2, preferred_element_type=jnp.float32)
    gb = jnp.broadcast_to(g_ref[0], (BLK, BLK))
    ir = lax.broadcasted_iota(jnp.int32, (BLK, BLK), 0)
    ic = lax.broadcasted_iota(jnp.int32, (BLK, BLK), 1)
    diag = jnp.where(ir == ic, gb, jnp.zeros_like(gb))
    y_ref[...] = jnp.dot(diag, y, preferred_element_type=jnp.float32)


def _grouped_mlp(block_meta, x_s, w_fc, w_proj, gates3):
    grid_spec = pltpu.PrefetchScalarGridSpec(
        num_scalar_prefetch=1,
        grid=(NB,),
        in_specs=[
            pl.BlockSpec((BLK, D2), lambda i, m: (i, 0)),
            pl.BlockSpec(memory_space=pl.ANY),
            pl.BlockSpec(memory_space=pl.ANY),
            pl.BlockSpec((1, 1, BLK), lambda i, m: (i, 0, 0)),
        ],
        out_specs=pl.BlockSpec((BLK, D), lambda i, m: (i, 0)),
        scratch_shapes=[
            pltpu.VMEM((2, D, F), jnp.float32),
            pltpu.VMEM((2, F, D), jnp.float32),
            pltpu.SemaphoreType.DMA((2, 2)),
        ],
    )
    return pl.pallas_call(
        _mlp_block_kernel,
        grid_spec=grid_spec,
        out_shape=jax.ShapeDtypeStruct((NP, D), jnp.float32),
        compiler_params=pltpu.CompilerParams(
            dimension_semantics=("arbitrary",),
        ),
    )(block_meta, x_s, w_fc, w_proj, gates3)


def _dispatch_meta(eids, gflat):
    onehot = (eids[None, :] == jnp.arange(E, dtype=jnp.int32)[:, None]).astype(
        jnp.int32
    )
    incl = jnp.cumsum(onehot, axis=1)
    counts = incl[:, -1]
    ranks = jnp.sum(incl * onehot, axis=0) - 1
    nblk = (counts + BLK - 1) // BLK
    off = jnp.concatenate(
        [jnp.zeros((1,), jnp.int32), jnp.cumsum(nblk * BLK)[:-1].astype(jnp.int32)]
    )
    off_pair = jnp.sum(off[:, None] * onehot, axis=0)
    pos = off_pair + ranks

    p_token = jnp.arange(P, dtype=jnp.int32) // K
    row_token = jnp.zeros((NP,), jnp.int32).at[pos].set(p_token)
    row_gate = jnp.zeros((NP,), jnp.float32).at[pos].set(gflat)
    be = jnp.repeat(jnp.arange(E, dtype=jnp.int32), nblk, total_repeat_length=NB)
    first = jnp.concatenate(
        [jnp.ones((1,), jnp.int32), (be[1:] != be[:-1]).astype(jnp.int32)]
    )
    run_id = jnp.cumsum(first) - 1
    par = (run_id % 2).astype(jnp.int32)
    idx_end = jnp.searchsorted(be, be, side="right").astype(jnp.int32)
    has_next = (idx_end < NB).astype(jnp.int32)
    nxt_e = be[jnp.minimum(idx_end, NB - 1)]
    block_meta = jnp.stack([be, par, first, has_next, nxt_e]).astype(jnp.int32)
    pos01 = pos.reshape(T, K).T.reshape(K, NW, NCH, CCH)
    gates3 = row_gate.reshape(NB, 1, BLK)
    return row_token, gates3, block_meta, pos01


def kernel(hidden_states, routing_weights, selected_experts, W_fc, W_proj):
    eids = selected_experts.reshape(-1).astype(jnp.int32)
    gflat = routing_weights.reshape(-1).astype(jnp.float32)
    row_token, gates3, block_meta, pos01 = _dispatch_meta(eids, gflat)

    hu = jax.lax.bitcast_convert_type(
        hidden_states.astype(jnp.bfloat16), jnp.uint16
    ).astype(jnp.uint32)
    h_pk = jax.lax.bitcast_convert_type(
        hu[:, :D2] | (hu[:, D2:] << 16), jnp.int32
    )
    x_s = _sc_gather_rows()(h_pk, row_token)
    y = _grouped_mlp(block_meta, x_s, W_fc, W_proj, gates3)
    out = _sc_combine()(y, pos01)
    return out
